# Initial kernel scaffold; baseline (speedup 1.0000x reference)
import jax
import jax.numpy as jnp
from jax.experimental import pallas as pl


def kernel(protein_x, protein_edge_index, drug_x, drug_edge_index, protein_seq, drug_seq, Wp1, bp1, Wp2, bp2, Wd1, bd1, Wd2, bd2, Wps1, bps1, Wps2, bps2, Wds1, bds1, Wds2, bds2, Wfc1, bfc1, Wfc2, bfc2):
    raise NotImplementedError("write your pallas kernel here")


# speedup vs baseline: 7.7016x; 7.7016x over previous
import functools

import jax
import jax.numpy as jnp
from jax import lax
from jax.experimental import pallas as pl
from jax.experimental.pallas import tpu as pltpu
from jax.experimental.pallas import tpu_sc as plsc

N = 10000
NPAD = 10240
D = 128
NC = 2
NS = 16
NW = NC * NS
CHUNK = 128
RPT = NPAD // NS


def _pad_edges(ei):
  src = ei[0].astype(jnp.int32)
  dst = ei[1].astype(jnp.int32)
  e = src.shape[0]
  nct = -(-e // (CHUNK * NW)) * NW
  epad = nct * CHUNK
  pad = jnp.full((epad - e,), N, jnp.int32)
  src1 = jnp.concatenate([src, pad])
  dst1 = jnp.concatenate([dst, pad])
  return src1, dst1, nct


def _mesh():
  return plsc.VectorSubcoreMesh(
      core_axis_name="c", subcore_axis_name="s",
      num_cores=NC, num_subcores=NS)


@functools.lru_cache(maxsize=None)
def _sc_kernel(nct, deg_mode):
  nc = nct // NW

  scratch = [
      pltpu.VMEM((CHUNK,), jnp.int32),
      pltpu.VMEM((CHUNK, D), jnp.float32),
      pltpu.VMEM_SHARED((NPAD, D), jnp.float32),
  ]
  if not deg_mode:
    scratch += [
        pltpu.VMEM((CHUNK,), jnp.int32),
        pltpu.SemaphoreType.DMA,
    ]

  @functools.partial(
      pl.kernel,
      out_type=jax.ShapeDtypeStruct((NC, NPAD, D), jnp.float32),
      mesh=_mesh(),
      scratch_types=scratch,
  )
  def sc_k(*args):
    if deg_mode:
      dst_hbm, out_hbm, dst_stage, rows, acc_sh = args
      src_hbm = src_stage = sem = None
    else:
      h2_hbm, src_hbm, dst_hbm, out_hbm, dst_stage, rows, acc_sh, src_stage, sem = args
    cid = lax.axis_index("c")
    sid = lax.axis_index("s")
    wid = cid * NS + sid

    def fill(val):
      def frow(i, c):
        for k in range(D // 16):
          rows[i, pl.ds(k * 16, 16)] = jnp.full((16,), val, jnp.float32)
        return c
      lax.fori_loop(0, CHUNK, frow, 0)

    fill(0.0)
    for k in range(RPT // CHUNK):
      pltpu.sync_copy(rows, acc_sh.at[pl.ds(sid * RPT + k * CHUNK, CHUNK)])
    if deg_mode:
      fill(1.0)
    plsc.subcore_barrier()

    def step(j, c):
      base = (wid * nc + j) * CHUNK
      pltpu.sync_copy(dst_hbm.at[pl.ds(base, CHUNK)], dst_stage)
      if not deg_mode:
        pltpu.sync_copy(src_hbm.at[pl.ds(base, CHUNK)], src_stage)
        pltpu.async_copy(h2_hbm.at[src_stage], rows, sem).wait()
      pltpu.sync_copy(rows, acc_sh.at[dst_stage], add=True)
      return c
    lax.fori_loop(0, nc, step, 0)

    plsc.subcore_barrier()
    pltpu.sync_copy(acc_sh.at[pl.ds(sid * RPT, RPT)],
                    out_hbm.at[cid, pl.ds(sid * RPT, RPT)])

  return sc_k


def _tc_prep(x_pad, deg_part, W1):
  def body(x_r, deg_r, w_r, h2_o, dis_o):
    valid = lax.broadcasted_iota(jnp.int32, (NPAD, D), 0) < N
    deg = deg_r[0] + deg_r[1] + 1.0
    dis = jnp.where(valid, lax.rsqrt(deg), 0.0)
    dis_o[...] = dis
    h = jnp.dot(x_r[...], w_r[...], preferred_element_type=jnp.float32)
    h2_o[...] = h * dis

  return pl.pallas_call(
      body,
      out_shape=(jax.ShapeDtypeStruct((NPAD, D), jnp.float32),
                 jax.ShapeDtypeStruct((NPAD, D), jnp.float32)),
  )(x_pad, deg_part, W1)


def _tc_mid(acc, h2, dis, b1, W2):
  def body(acc_r, h2_r, dis_r, b_r, w_r, h2b_o):
    dis_v = dis_r[...]
    y = jnp.maximum(dis_v * (acc_r[0] + acc_r[1] + h2_r[...]) + b_r[...], 0.0)
    h2b_o[...] = jnp.dot(y, w_r[...], preferred_element_type=jnp.float32) * dis_v

  return pl.pallas_call(
      body,
      out_shape=jax.ShapeDtypeStruct((NPAD, D), jnp.float32),
  )(acc, h2, dis, b1, W2)


def _tc_pool(acc, h2, dis, b2):
  def body(acc_r, h2_r, dis_r, b_r, g_o):
    y = jnp.maximum(dis_r[...] * (acc_r[0] + acc_r[1] + h2_r[...]) + b_r[...],
                    0.0)
    mask = lax.broadcasted_iota(jnp.int32, (NPAD, D), 0) < N
    y = jnp.where(mask, y, 0.0)
    g_o[...] = jnp.sum(y, axis=0, keepdims=True) * (1.0 / N)

  return pl.pallas_call(
      body,
      out_shape=jax.ShapeDtypeStruct((1, D), jnp.float32),
  )(acc, h2, dis, b2)


def _tc_head(pg, dg, psq, dsq, Wps1, bps1, Wps2, bps2,
             Wds1, bds1, Wds2, bds2, Wfc1, bfc1, Wfc2, bfc2):
  def body(pg_r, dg_r, psq_r, dsq_r, wps1, b1p, wps2, b2p,
           wds1, b1d, wds2, b2d, wf1, bf1, wf2, bf2, out_o):
    def enc(s, w1, b1, w2, b2):
      h = jnp.maximum(jnp.dot(s, w1[...], preferred_element_type=jnp.float32)
                      + b1[...], 0.0)
      return jnp.maximum(jnp.dot(h, w2[...], preferred_element_type=jnp.float32)
                         + b2[...], 0.0)
    ps = enc(psq_r[...], wps1, b1p, wps2, b2p)
    ds = enc(dsq_r[...], wds1, b1d, wds2, b2d)
    comb = jnp.concatenate([pg_r[...], dg_r[...], ps, ds], axis=1)
    c1 = jnp.maximum(jnp.dot(comb, wf1[...], preferred_element_type=jnp.float32)
                     + bf1[...], 0.0)
    out_o[...] = (jnp.dot(c1, wf2[...], preferred_element_type=jnp.float32)
                  + bf2[...])

  return pl.pallas_call(
      body,
      out_shape=jax.ShapeDtypeStruct((1, 1), jnp.float32),
  )(pg, dg, psq, dsq, Wps1, bps1.reshape(1, -1), Wps2, bps2.reshape(1, -1),
    Wds1, bds1.reshape(1, -1), Wds2, bds2.reshape(1, -1),
    Wfc1, bfc1.reshape(1, -1), Wfc2, bfc2.reshape(1, -1))


def _graph_branch(x, ei, W1, b1, W2, b2):
  src1, dst1, nct = _pad_edges(ei)
  x_pad = jnp.zeros((NPAD, D), jnp.float32).at[:N].set(x)
  deg_part = _sc_kernel(nct, True)(dst1)
  h2, dis = _tc_prep(x_pad, deg_part, W1)
  acc1 = _sc_kernel(nct, False)(h2, src1, dst1)
  h2b = _tc_mid(acc1, h2, dis, b1.reshape(1, -1), W2)
  acc2 = _sc_kernel(nct, False)(h2b, src1, dst1)
  return _tc_pool(acc2, h2b, dis, b2.reshape(1, -1))


def kernel(protein_x, protein_edge_index, drug_x, drug_edge_index,
           protein_seq, drug_seq,
           Wp1, bp1, Wp2, bp2, Wd1, bd1, Wd2, bd2,
           Wps1, bps1, Wps2, bps2, Wds1, bds1, Wds2, bds2,
           Wfc1, bfc1, Wfc2, bfc2):
  pg = _graph_branch(protein_x, protein_edge_index, Wp1, bp1, Wp2, bp2)
  dg = _graph_branch(drug_x, drug_edge_index, Wd1, bd1, Wd2, bd2)
  out = _tc_head(pg, dg, protein_seq.reshape(1, -1), drug_seq.reshape(1, -1),
                 Wps1, bps1, Wps2, bps2, Wds1, bds1, Wds2, bds2,
                 Wfc1, bfc1, Wfc2, bfc2)
  return out.reshape(1)
